# baseline (speedup 1.0000x reference)
import jax
import jax.numpy as jnp
from jax import lax
from jax.experimental import pallas as pl
from jax.experimental.pallas import tpu as pltpu
from jax.experimental.pallas import tpu_sc as plsc

NUM_CLASS = 100000
CTX_DIM = 512
TOK_LEN = 77
MAX_LENGTH = 77
N_PREFIX = 4
N_DYN = MAX_LENGTH - N_PREFIX
WIN = 80
BATCH = 1024
NUM_WORKERS = 32
PER_W = BATCH // NUM_WORKERS
ROWS_PER_STEP = 16


def _tc_token_gather(labels, class_tokens):
    def body(labels_ref, *refs):
        out_ref = refs[-1]
        sub = lax.broadcasted_iota(jnp.int32, (8, TOK_LEN), 0)
        for j in range(ROWS_PER_STEP):
            lab = labels_ref[pl.program_id(0) * ROWS_PER_STEP + j]
            r = lab % 8
            row = jnp.sum(jnp.where(sub == r, refs[j][...], 0),
                          axis=0, keepdims=True)
            out_ref[pl.ds(j, 1), :] = row

    def in_map(j):
        return lambda i, lr: (lr[ROWS_PER_STEP * i + j] // 8, 0)

    tokens = pl.pallas_call(
        body,
        grid_spec=pltpu.PrefetchScalarGridSpec(
            num_scalar_prefetch=1,
            grid=(BATCH // ROWS_PER_STEP,),
            in_specs=[pl.BlockSpec((8, TOK_LEN), in_map(j))
                      for j in range(ROWS_PER_STEP)],
            out_specs=pl.BlockSpec((ROWS_PER_STEP, TOK_LEN),
                                   lambda i, lr: (i, 0)),
        ),
        out_shape=jax.ShapeDtypeStruct((BATCH, TOK_LEN), jnp.int32),
    )(labels, *([class_tokens] * ROWS_PER_STEP))
    return tokens.reshape(-1)


def _sc_body(tok1d_hbm, table_hbm, prefix_hbm, out_hbm,
             tokf_v, bufg0, bufg1, pbuf, gsem0, gsem1, ssem):
    wid = lax.axis_index("s") * 2 + lax.axis_index("c")
    base = wid * PER_W

    pltpu.sync_copy(tok1d_hbm.at[pl.ds(base * TOK_LEN, PER_W * TOK_LEN)],
                    tokf_v)
    pltpu.sync_copy(prefix_hbm.at[0], pbuf)

    bufs = (bufg0, bufg1)
    gsems = (gsem0, gsem1)

    def gather(i, b):
        start = TOK_LEN * i
        a = pl.multiple_of((start >> 3) << 3, 8)
        dd = start - a
        idx = tokf_v.at[pl.ds(a, WIN)]
        h = pltpu.async_copy(table_hbm.at[idx], bufs[b], gsems[b])
        return h, dd

    def store(i, b, dd):
        pltpu.sync_copy(pbuf, out_hbm.at[base + i, pl.ds(0, N_PREFIX)])
        return pltpu.async_copy(bufs[b].at[pl.ds(dd, N_DYN)],
                                out_hbm.at[base + i, pl.ds(N_PREFIX, N_DYN)],
                                ssem)

    def body(p, carry):
        i0 = 2 * p
        h0, dd0 = gather(i0, 0)
        h1, dd1 = gather(i0 + 1, 1)
        h0.wait()
        s0 = store(i0, 0, dd0)
        h1.wait()
        s1 = store(i0 + 1, 1, dd1)
        s0.wait()
        s1.wait()
        return carry

    lax.fori_loop(0, PER_W // 2, body, 0)


def kernel(labels, token_embedding_table, token_prefix, token_suffix,
           class_tokens):
    del token_suffix
    tok1d = _tc_token_gather(labels, class_tokens)

    mesh = plsc.VectorSubcoreMesh(core_axis_name="c", subcore_axis_name="s")
    kfn = pl.kernel(
        _sc_body,
        mesh=mesh,
        compiler_params=pltpu.CompilerParams(use_tc_tiling_on_sc=False),
        out_type=jax.ShapeDtypeStruct((BATCH, MAX_LENGTH, CTX_DIM),
                                      jnp.float32),
        scratch_types=[
            pltpu.VMEM((PER_W * TOK_LEN,), jnp.int32),
            pltpu.VMEM((WIN, CTX_DIM), jnp.float32),
            pltpu.VMEM((WIN, CTX_DIM), jnp.float32),
            pltpu.VMEM((N_PREFIX, CTX_DIM), jnp.float32),
            pltpu.SemaphoreType.DMA,
            pltpu.SemaphoreType.DMA,
            pltpu.SemaphoreType.DMA,
        ],
    )
    return kfn(tok1d, token_embedding_table, token_prefix)


# speedup vs baseline: 1.0217x; 1.0217x over previous
import jax
import jax.numpy as jnp
from jax import lax
from jax.experimental import pallas as pl
from jax.experimental.pallas import tpu as pltpu
from jax.experimental.pallas import tpu_sc as plsc

NUM_CLASS = 100000
CTX_DIM = 512
TOK_LEN = 77
MAX_LENGTH = 77
N_PREFIX = 4
N_DYN = MAX_LENGTH - N_PREFIX
WIN = 80
BATCH = 1024
NUM_WORKERS = 32
PER_W = BATCH // NUM_WORKERS
ROWS_PER_STEP = 32


def _tc_token_gather(labels, class_tokens):
    def body(labels_ref, *refs):
        out_ref = refs[-1]
        sub = lax.broadcasted_iota(jnp.int32, (8, TOK_LEN), 0)
        for j in range(ROWS_PER_STEP):
            lab = labels_ref[pl.program_id(0) * ROWS_PER_STEP + j]
            r = lab % 8
            row = jnp.sum(jnp.where(sub == r, refs[j][...], 0),
                          axis=0, keepdims=True)
            out_ref[pl.ds(j, 1), :] = row

    def in_map(j):
        return lambda i, lr: (lr[ROWS_PER_STEP * i + j] // 8, 0)

    tokens = pl.pallas_call(
        body,
        grid_spec=pltpu.PrefetchScalarGridSpec(
            num_scalar_prefetch=1,
            grid=(BATCH // ROWS_PER_STEP,),
            in_specs=[pl.BlockSpec((8, TOK_LEN), in_map(j))
                      for j in range(ROWS_PER_STEP)],
            out_specs=pl.BlockSpec((ROWS_PER_STEP, TOK_LEN),
                                   lambda i, lr: (i, 0)),
        ),
        out_shape=jax.ShapeDtypeStruct((BATCH, TOK_LEN), jnp.int32),
    )(labels, *([class_tokens] * ROWS_PER_STEP))
    return tokens.reshape(-1)


def _sc_body(tok1d_hbm, table_hbm, prefix_hbm, out_hbm,
             tokf_v, bufg0, bufg1, pbuf, gsem0, gsem1, ssem):
    wid = lax.axis_index("s") * 2 + lax.axis_index("c")
    base = wid * PER_W

    pltpu.sync_copy(tok1d_hbm.at[pl.ds(base * TOK_LEN, PER_W * TOK_LEN)],
                    tokf_v)
    pltpu.sync_copy(prefix_hbm.at[0], pbuf)

    bufs = (bufg0, bufg1)
    gsems = (gsem0, gsem1)

    def gather(i, b):
        start = TOK_LEN * i
        a = pl.multiple_of((start >> 3) << 3, 8)
        dd = start - a
        idx = tokf_v.at[pl.ds(a, WIN)]
        h = pltpu.async_copy(table_hbm.at[idx], bufs[b], gsems[b])
        return h, dd

    def store(i, b, dd):
        pltpu.sync_copy(pbuf, out_hbm.at[base + i, pl.ds(0, N_PREFIX)])
        return pltpu.async_copy(bufs[b].at[pl.ds(dd, N_DYN)],
                                out_hbm.at[base + i, pl.ds(N_PREFIX, N_DYN)],
                                ssem)

    def body(p, carry):
        i0 = 2 * p
        h0, dd0 = gather(i0, 0)
        h1, dd1 = gather(i0 + 1, 1)
        h0.wait()
        s0 = store(i0, 0, dd0)
        h1.wait()
        s1 = store(i0 + 1, 1, dd1)
        s0.wait()
        s1.wait()
        return carry

    lax.fori_loop(0, PER_W // 2, body, 0)


def kernel(labels, token_embedding_table, token_prefix, token_suffix,
           class_tokens):
    del token_suffix
    tok1d = _tc_token_gather(labels, class_tokens)

    mesh = plsc.VectorSubcoreMesh(core_axis_name="c", subcore_axis_name="s")
    kfn = pl.kernel(
        _sc_body,
        mesh=mesh,
        compiler_params=pltpu.CompilerParams(use_tc_tiling_on_sc=False),
        out_type=jax.ShapeDtypeStruct((BATCH, MAX_LENGTH, CTX_DIM),
                                      jnp.float32),
        scratch_types=[
            pltpu.VMEM((PER_W * TOK_LEN,), jnp.int32),
            pltpu.VMEM((WIN, CTX_DIM), jnp.float32),
            pltpu.VMEM((WIN, CTX_DIM), jnp.float32),
            pltpu.VMEM((N_PREFIX, CTX_DIM), jnp.float32),
            pltpu.SemaphoreType.DMA,
            pltpu.SemaphoreType.DMA,
            pltpu.SemaphoreType.DMA,
        ],
    )
    return kfn(tok1d, token_embedding_table, token_prefix)


# speedup vs baseline: 1.0323x; 1.0103x over previous
import jax
import jax.numpy as jnp
from jax import lax
from jax.experimental import pallas as pl
from jax.experimental.pallas import tpu as pltpu
from jax.experimental.pallas import tpu_sc as plsc

NUM_CLASS = 100000
CTX_DIM = 512
TOK_LEN = 77
MAX_LENGTH = 77
N_PREFIX = 4
N_DYN = MAX_LENGTH - N_PREFIX
WIN = 80
BATCH = 1024
NUM_WORKERS = 32
PER_W = BATCH // NUM_WORKERS
ROWS_PER_STEP = 64


def _tc_token_gather(labels, class_tokens):
    def body(labels_ref, *refs):
        out_ref = refs[-1]
        sub = lax.broadcasted_iota(jnp.int32, (8, TOK_LEN), 0)
        for j in range(ROWS_PER_STEP):
            lab = labels_ref[pl.program_id(0) * ROWS_PER_STEP + j]
            r = lab % 8
            row = jnp.sum(jnp.where(sub == r, refs[j][...], 0),
                          axis=0, keepdims=True)
            out_ref[pl.ds(j, 1), :] = row

    def in_map(j):
        return lambda i, lr: (lr[ROWS_PER_STEP * i + j] // 8, 0)

    tokens = pl.pallas_call(
        body,
        grid_spec=pltpu.PrefetchScalarGridSpec(
            num_scalar_prefetch=1,
            grid=(BATCH // ROWS_PER_STEP,),
            in_specs=[pl.BlockSpec((8, TOK_LEN), in_map(j))
                      for j in range(ROWS_PER_STEP)],
            out_specs=pl.BlockSpec((ROWS_PER_STEP, TOK_LEN),
                                   lambda i, lr: (i, 0)),
        ),
        out_shape=jax.ShapeDtypeStruct((BATCH, TOK_LEN), jnp.int32),
    )(labels, *([class_tokens] * ROWS_PER_STEP))
    return tokens.reshape(-1)


def _sc_body(tok1d_hbm, table_hbm, prefix_hbm, out_hbm,
             tokf_v, bufg0, bufg1, pbuf, gsem0, gsem1, ssem):
    wid = lax.axis_index("s") * 2 + lax.axis_index("c")
    base = wid * PER_W

    pltpu.sync_copy(tok1d_hbm.at[pl.ds(base * TOK_LEN, PER_W * TOK_LEN)],
                    tokf_v)
    pltpu.sync_copy(prefix_hbm.at[0], pbuf)

    bufs = (bufg0, bufg1)
    gsems = (gsem0, gsem1)

    def gather(i, b):
        start = TOK_LEN * i
        a = pl.multiple_of((start >> 3) << 3, 8)
        dd = start - a
        idx = tokf_v.at[pl.ds(a, WIN)]
        h = pltpu.async_copy(table_hbm.at[idx], bufs[b], gsems[b])
        return h, dd

    def store(i, b, dd):
        pltpu.sync_copy(pbuf, out_hbm.at[base + i, pl.ds(0, N_PREFIX)])
        return pltpu.async_copy(bufs[b].at[pl.ds(dd, N_DYN)],
                                out_hbm.at[base + i, pl.ds(N_PREFIX, N_DYN)],
                                ssem)

    def body(p, carry):
        i0 = 2 * p
        h0, dd0 = gather(i0, 0)
        h1, dd1 = gather(i0 + 1, 1)
        h0.wait()
        s0 = store(i0, 0, dd0)
        h1.wait()
        s1 = store(i0 + 1, 1, dd1)
        s0.wait()
        s1.wait()
        return carry

    lax.fori_loop(0, PER_W // 2, body, 0)


def kernel(labels, token_embedding_table, token_prefix, token_suffix,
           class_tokens):
    del token_suffix
    tok1d = _tc_token_gather(labels, class_tokens)

    mesh = plsc.VectorSubcoreMesh(core_axis_name="c", subcore_axis_name="s")
    kfn = pl.kernel(
        _sc_body,
        mesh=mesh,
        compiler_params=pltpu.CompilerParams(use_tc_tiling_on_sc=False),
        out_type=jax.ShapeDtypeStruct((BATCH, MAX_LENGTH, CTX_DIM),
                                      jnp.float32),
        scratch_types=[
            pltpu.VMEM((PER_W * TOK_LEN,), jnp.int32),
            pltpu.VMEM((WIN, CTX_DIM), jnp.float32),
            pltpu.VMEM((WIN, CTX_DIM), jnp.float32),
            pltpu.VMEM((N_PREFIX, CTX_DIM), jnp.float32),
            pltpu.SemaphoreType.DMA,
            pltpu.SemaphoreType.DMA,
            pltpu.SemaphoreType.DMA,
        ],
    )
    return kfn(tok1d, token_embedding_table, token_prefix)
